# baseline (speedup 1.0000x reference)
import functools
import math

import jax
import jax.numpy as jnp
from jax import lax
from jax.experimental import pallas as pl
from jax.experimental.pallas import tpu as pltpu
from jax.experimental.pallas import tpu_sc as plsc

N_BINS = 8
LANES = 16
NC = 2
NS = 16
NW = NC * NS
CHUNK = 4096

import numpy as np

_TWO_PI = np.float32(2.0 * math.pi)
_INV_BIN = np.float32(N_BINS / (2.0 * math.pi))

_MESH = plsc.VectorSubcoreMesh(core_axis_name="c", subcore_axis_name="s")
_SC_PARAMS = pltpu.CompilerParams(needs_layout_passes=False)


def _worker_id():
    return lax.axis_index("s") * NC + lax.axis_index("c")


def _make_min_kernel(n):
    per_w = n // NW
    nchunks = per_w // CHUNK

    @functools.partial(
        pl.kernel,
        out_type=jax.ShapeDtypeStruct((NW, LANES), jnp.float32),
        mesh=_MESH,
        scratch_types=[
            pltpu.VMEM((CHUNK,), jnp.float32),
            pltpu.VMEM((LANES,), jnp.float32),
        ],
        compiler_params=_SC_PARAMS,
    )
    def min_kernel(x_hbm, out_hbm, xbuf, minbuf):
        wid = _worker_id()
        base = wid * per_w

        def chunk_body(ci, acc):
            pltpu.sync_copy(x_hbm.at[pl.ds(base + ci * CHUNK, CHUNK)], xbuf)

            def vbody(i, a):
                return jnp.minimum(a, xbuf[pl.ds(i * LANES, LANES)])

            return lax.fori_loop(0, CHUNK // LANES, vbody, acc)

        acc = lax.fori_loop(
            0, nchunks, chunk_body, jnp.full((LANES,), jnp.inf, jnp.float32)
        )
        minbuf[...] = acc
        pltpu.sync_copy(minbuf, out_hbm.at[wid])

    return min_kernel


def _make_encode_kernel(n):
    per_w = n // NW
    nchunks = per_w // CHUNK

    @functools.partial(
        pl.kernel,
        out_type=jax.ShapeDtypeStruct((n * N_BINS,), jnp.float32),
        mesh=_MESH,
        scratch_types=[
            pltpu.VMEM((CHUNK,), jnp.float32),
            pltpu.VMEM((CHUNK * N_BINS,), jnp.float32),
            pltpu.VMEM((NW, LANES), jnp.float32),
            pltpu.VMEM((LANES,), jnp.float32),
        ],
        compiler_params=_SC_PARAMS,
    )
    def encode_kernel(x_hbm, pmin_hbm, out_hbm, xbuf, obuf, pbuf, mbuf):
        wid = _worker_id()
        base = wid * per_w

        pltpu.sync_copy(pmin_hbm, pbuf)

        def mb(i, a):
            return jnp.minimum(a, pbuf[i])

        acc = lax.fori_loop(0, NW, mb, jnp.full((LANES,), jnp.inf, jnp.float32))
        lane8 = lax.iota(jnp.int32, LANES) * N_BINS
        ones = jnp.full((LANES,), 1.0, jnp.float32)
        zeros = jnp.zeros((LANES,), jnp.float32)
        lane = lax.iota(jnp.int32, LANES)
        for stride in (8, 4, 2, 1):
            mbuf[...] = acc
            acc = jnp.minimum(acc, plsc.load_gather(mbuf, [lane ^ stride]))
        off = jnp.where(acc < 0.0, ones, zeros)

        def chunk_body(ci, _):
            src = base + ci * CHUNK
            pltpu.sync_copy(x_hbm.at[pl.ds(src, CHUNK)], xbuf)

            def vbody(i, __):
                v = xbuf[pl.ds(i * LANES, LANES)]
                xn = v + off
                r = lax.rem(xn, _TWO_PI)
                r = jnp.where(r < 0.0, r + _TWO_PI, r)
                b = jnp.minimum((r * _INV_BIN).astype(jnp.int32), N_BINS - 1)
                valid = r < _TWO_PI
                obase = i * (LANES * N_BINS)
                for k in range(N_BINS):
                    obuf[pl.ds(obase + k * LANES, LANES)] = zeros
                plsc.store_scatter(obuf, [lane8 + (b + obase)], ones, mask=valid)
                return 0

            lax.fori_loop(0, CHUNK // LANES, vbody, 0)
            pltpu.sync_copy(obuf, out_hbm.at[pl.ds(src * N_BINS, CHUNK * N_BINS)])
            return 0

        lax.fori_loop(0, nchunks, chunk_body, 0)

    return encode_kernel


def kernel(x, phase_bins):
    del phase_bins
    n = x.size
    xf = x.reshape(-1)
    pmin = _make_min_kernel(n)(xf)
    outf = _make_encode_kernel(n)(xf, pmin)
    return outf.reshape(x.shape + (N_BINS,))


# speedup vs baseline: 7.6037x; 7.6037x over previous
import functools
import math

import jax
import jax.numpy as jnp
import numpy as np
from jax import lax
from jax.experimental import pallas as pl
from jax.experimental.pallas import tpu as pltpu
from jax.experimental.pallas import tpu_sc as plsc

N_BINS = 8
LANES = 16
NC = 2
NS = 16
NW = NC * NS

_TWO_PI = np.float32(2.0 * math.pi)
_INV_BIN = np.float32(N_BINS / (2.0 * math.pi))

_MESH = plsc.VectorSubcoreMesh(core_axis_name="c", subcore_axis_name="s")
_SC_PARAMS = pltpu.CompilerParams(needs_layout_passes=False)


def _worker_id():
    return lax.axis_index("s") * NC + lax.axis_index("c")


def _make_min_kernel(n):
    per_w = n // NW
    chunk = 8192
    nch = per_w // chunk

    @functools.partial(
        pl.kernel,
        out_type=jax.ShapeDtypeStruct((NW, LANES), jnp.float32),
        mesh=_MESH,
        scratch_types=[
            pltpu.VMEM((chunk,), jnp.float32),
            pltpu.VMEM((chunk,), jnp.float32),
            pltpu.VMEM((LANES,), jnp.float32),
            pltpu.SemaphoreType.DMA,
            pltpu.SemaphoreType.DMA,
        ],
        compiler_params=_SC_PARAMS,
    )
    def min_kernel(x_hbm, out_hbm, xb0, xb1, minbuf, s0, s1):
        wid = _worker_id()
        base = wid * per_w
        xbufs = (xb0, xb1)
        sems = (s0, s1)

        def in_slice(c):
            return x_hbm.at[pl.ds(base + c * chunk, chunk)]

        pltpu.async_copy(in_slice(0), xb0, s0)
        pltpu.async_copy(in_slice(1), xb1, s1)

        def gbody(g, acc):
            for b in range(2):
                c = 2 * g + b
                pltpu.make_async_copy(in_slice(c), xbufs[b], sems[b]).wait()

                def vbody(i, a):
                    return jnp.minimum(a, xbufs[b][pl.ds(i * LANES, LANES)])

                acc = lax.fori_loop(0, chunk // LANES, vbody, acc, unroll=8)

                @pl.when(c + 2 < nch)
                def _():
                    pltpu.async_copy(in_slice(c + 2), xbufs[b], sems[b])
            return acc

        acc = lax.fori_loop(
            0, nch // 2, gbody, jnp.full((LANES,), jnp.inf, jnp.float32)
        )
        minbuf[...] = acc
        pltpu.sync_copy(minbuf, out_hbm.at[wid])

    return min_kernel


def _make_encode_kernel(n_rows, n_cols):
    rows_per_w = n_rows // NW
    rch = 2
    cw = rch * n_cols
    ow = cw * N_BINS
    nch = rows_per_w // rch
    n_tiles = n_cols // 128

    @functools.partial(
        pl.kernel,
        out_type=jax.ShapeDtypeStruct((n_rows * n_cols * N_BINS,), jnp.float32),
        mesh=_MESH,
        scratch_types=[
            pltpu.VMEM((cw,), jnp.float32),
            pltpu.VMEM((cw,), jnp.float32),
            pltpu.VMEM((ow,), jnp.float32),
            pltpu.VMEM((ow,), jnp.float32),
            pltpu.VMEM((NW, LANES), jnp.float32),
            pltpu.VMEM((LANES,), jnp.float32),
            pltpu.SemaphoreType.DMA,
            pltpu.SemaphoreType.DMA,
            pltpu.SemaphoreType.DMA,
            pltpu.SemaphoreType.DMA,
        ],
        compiler_params=_SC_PARAMS,
    )
    def encode_kernel(x_hbm, pmin_hbm, out_hbm,
                      xb0, xb1, ob0, ob1, pbuf, mbuf, si0, si1, so0, so1):
        wid = _worker_id()
        in_base = wid * rows_per_w * n_cols
        out_base = in_base * N_BINS
        xbufs = (xb0, xb1)
        obufs = (ob0, ob1)
        isems = (si0, si1)
        osems = (so0, so1)

        pltpu.sync_copy(pmin_hbm, pbuf)

        def mb(i, a):
            return jnp.minimum(a, pbuf[i])

        acc = lax.fori_loop(0, NW, mb, jnp.full((LANES,), jnp.inf, jnp.float32))
        lane = lax.iota(jnp.int32, LANES)
        ones = jnp.full((LANES,), 1.0, jnp.float32)
        zeros = jnp.zeros((LANES,), jnp.float32)
        for stride in (8, 4, 2, 1):
            mbuf[...] = acc
            acc = jnp.minimum(acc, plsc.load_gather(mbuf, [lane ^ stride]))
        off = jnp.where(acc < 0.0, ones, zeros)
        eight = jnp.full((LANES,), N_BINS, jnp.int32)

        def in_slice(c):
            return x_hbm.at[pl.ds(in_base + c * cw, cw)]

        def out_slice(c):
            return out_hbm.at[pl.ds(out_base + c * ow, ow)]

        pltpu.async_copy(in_slice(0), xb0, si0)
        pltpu.async_copy(in_slice(1), xb1, si1)

        def compute(xb, ob):
            def vbody(v, _):
                xv = xb[pl.ds(v * LANES, LANES)]
                xn = xv + off
                r0 = lax.rem(xn, _TWO_PI)
                rr = jnp.where(r0 < 0.0, r0 + _TWO_PI, r0)
                bv = jnp.minimum((rr * _INV_BIN).astype(jnp.int32), N_BINS - 1)
                bv = jnp.where(rr < _TWO_PI, bv, eight)
                ob_base = ((v >> 3) << 10) + ((v & 7) << 4)
                for k in range(N_BINS):
                    mk = jnp.where(bv == k, ones, zeros)
                    ob[pl.ds(ob_base + k * 128, LANES)] = mk
                return 0

            lax.fori_loop(0, cw // LANES, vbody, 0)

        def gbody(g, _):
            for b in range(2):
                c = 2 * g + b
                pltpu.make_async_copy(in_slice(c), xbufs[b], isems[b]).wait()

                @pl.when(g > 0)
                def _():
                    pltpu.make_async_copy(obufs[b], out_slice(c), osems[b]).wait()

                compute(xbufs[b], obufs[b])
                pltpu.async_copy(obufs[b], out_slice(c), osems[b])

                @pl.when(c + 2 < nch)
                def _():
                    pltpu.async_copy(in_slice(c + 2), xbufs[b], isems[b])
            return 0

        lax.fori_loop(0, nch // 2, gbody, 0)
        pltpu.make_async_copy(ob0, out_slice(0), so0).wait()
        pltpu.make_async_copy(ob1, out_slice(1), so1).wait()

    return encode_kernel


def kernel(x, phase_bins):
    del phase_bins
    n_rows, n_cols = x.shape
    xf = x.reshape(-1)
    pmin = _make_min_kernel(x.size)(xf)
    outf = _make_encode_kernel(n_rows, n_cols)(xf, pmin)
    out4 = outf.reshape(n_rows, n_cols // 128, N_BINS, 128)
    return out4.transpose(0, 1, 3, 2).reshape(n_rows, n_cols, N_BINS)


# speedup vs baseline: 8.2367x; 1.0833x over previous
import functools
import math

import jax
import jax.numpy as jnp
import numpy as np
from jax import lax
from jax.experimental import pallas as pl
from jax.experimental.pallas import tpu as pltpu
from jax.experimental.pallas import tpu_sc as plsc

N_BINS = 8
LANES = 16
NC = 2
NS = 16
NW = NC * NS

_TWO_PI = np.float32(2.0 * math.pi)
_INV_BIN = np.float32(N_BINS / (2.0 * math.pi))

_MESH = plsc.VectorSubcoreMesh(core_axis_name="c", subcore_axis_name="s")
_SC_PARAMS = pltpu.CompilerParams(needs_layout_passes=False)


def _worker_id():
    return lax.axis_index("s") * NC + lax.axis_index("c")


def _make_min_kernel(n):
    per_w = n // NW
    chunk = 8192
    nch = per_w // chunk

    @functools.partial(
        pl.kernel,
        out_type=jax.ShapeDtypeStruct((NW, LANES), jnp.float32),
        mesh=_MESH,
        scratch_types=[
            pltpu.VMEM((chunk,), jnp.float32),
            pltpu.VMEM((chunk,), jnp.float32),
            pltpu.VMEM((LANES,), jnp.float32),
            pltpu.SemaphoreType.DMA,
            pltpu.SemaphoreType.DMA,
        ],
        compiler_params=_SC_PARAMS,
    )
    def min_kernel(x_hbm, out_hbm, xb0, xb1, minbuf, s0, s1):
        wid = _worker_id()
        base = wid * per_w
        xbufs = (xb0, xb1)
        sems = (s0, s1)

        def in_slice(c):
            return x_hbm.at[pl.ds(base + c * chunk, chunk)]

        pltpu.async_copy(in_slice(0), xb0, s0)
        pltpu.async_copy(in_slice(1), xb1, s1)

        def gbody(g, acc):
            for b in range(2):
                c = 2 * g + b
                pltpu.make_async_copy(in_slice(c), xbufs[b], sems[b]).wait()

                def vbody(i, a):
                    return jnp.minimum(a, xbufs[b][pl.ds(i * LANES, LANES)])

                acc = lax.fori_loop(0, chunk // LANES, vbody, acc, unroll=8)

                @pl.when(c + 2 < nch)
                def _():
                    pltpu.async_copy(in_slice(c + 2), xbufs[b], sems[b])
            return acc

        acc = lax.fori_loop(
            0, nch // 2, gbody, jnp.full((LANES,), jnp.inf, jnp.float32)
        )
        minbuf[...] = acc
        pltpu.sync_copy(minbuf, out_hbm.at[wid])

    return min_kernel


def _make_encode_kernel(n_rows, n_cols):
    rows_per_w = n_rows // NW
    rch = 2
    cw = rch * n_cols
    ow = cw * N_BINS
    nch = rows_per_w // rch
    n_tiles = n_cols // 128

    @functools.partial(
        pl.kernel,
        out_type=jax.ShapeDtypeStruct((n_rows * n_cols * N_BINS,), jnp.float32),
        mesh=_MESH,
        scratch_types=[
            pltpu.VMEM((cw,), jnp.float32),
            pltpu.VMEM((cw,), jnp.float32),
            pltpu.VMEM((ow,), jnp.float32),
            pltpu.VMEM((ow,), jnp.float32),
            pltpu.VMEM((NW, LANES), jnp.float32),
            pltpu.VMEM((LANES,), jnp.float32),
            pltpu.SemaphoreType.DMA,
            pltpu.SemaphoreType.DMA,
            pltpu.SemaphoreType.DMA,
            pltpu.SemaphoreType.DMA,
        ],
        compiler_params=_SC_PARAMS,
    )
    def encode_kernel(x_hbm, pmin_hbm, out_hbm,
                      xb0, xb1, ob0, ob1, pbuf, mbuf, si0, si1, so0, so1):
        wid = _worker_id()
        in_base = wid * rows_per_w * n_cols
        out_base = in_base * N_BINS
        xbufs = (xb0, xb1)
        obufs = (ob0, ob1)
        isems = (si0, si1)
        osems = (so0, so1)

        pltpu.sync_copy(pmin_hbm, pbuf)

        def mb(i, a):
            return jnp.minimum(a, pbuf[i])

        acc = lax.fori_loop(0, NW, mb, jnp.full((LANES,), jnp.inf, jnp.float32))
        lane = lax.iota(jnp.int32, LANES)
        ones = jnp.full((LANES,), 1.0, jnp.float32)
        zeros = jnp.zeros((LANES,), jnp.float32)
        for stride in (8, 4, 2, 1):
            mbuf[...] = acc
            acc = jnp.minimum(acc, plsc.load_gather(mbuf, [lane ^ stride]))
        off = jnp.where(acc < 0.0, ones, zeros)
        eight = jnp.full((LANES,), N_BINS, jnp.int32)

        def in_slice(c):
            return x_hbm.at[pl.ds(in_base + c * cw, cw)]

        def out_slice(c):
            return out_hbm.at[pl.ds(out_base + c * ow, ow)]

        pltpu.async_copy(in_slice(0), xb0, si0)
        pltpu.async_copy(in_slice(1), xb1, si1)

        def compute(xb, ob):
            @plsc.parallel_loop(0, cw // LANES, unroll=8)
            def _(v):
                xv = xb[pl.ds(v * LANES, LANES)]
                xn = xv + off
                r0 = lax.rem(xn, _TWO_PI)
                rr = jnp.where(r0 < 0.0, r0 + _TWO_PI, r0)
                bv = jnp.minimum((rr * _INV_BIN).astype(jnp.int32), N_BINS - 1)
                bv = jnp.where(rr < _TWO_PI, bv, eight)
                ob_base = ((v >> 3) << 10) + ((v & 7) << 4)
                for k in range(N_BINS):
                    mk = jnp.where(bv == k, ones, zeros)
                    ob[pl.ds(ob_base + k * 128, LANES)] = mk

        def gbody(g, _):
            for b in range(2):
                c = 2 * g + b
                pltpu.make_async_copy(in_slice(c), xbufs[b], isems[b]).wait()

                @pl.when(g > 0)
                def _():
                    pltpu.make_async_copy(obufs[b], out_slice(c), osems[b]).wait()

                compute(xbufs[b], obufs[b])
                pltpu.async_copy(obufs[b], out_slice(c), osems[b])

                @pl.when(c + 2 < nch)
                def _():
                    pltpu.async_copy(in_slice(c + 2), xbufs[b], isems[b])
            return 0

        lax.fori_loop(0, nch // 2, gbody, 0)
        pltpu.make_async_copy(ob0, out_slice(0), so0).wait()
        pltpu.make_async_copy(ob1, out_slice(1), so1).wait()

    return encode_kernel


def kernel(x, phase_bins):
    del phase_bins
    n_rows, n_cols = x.shape
    xf = x.reshape(-1)
    pmin = _make_min_kernel(x.size)(xf)
    outf = _make_encode_kernel(n_rows, n_cols)(xf, pmin)
    out4 = outf.reshape(n_rows, n_cols // 128, N_BINS, 128)
    return out4.transpose(0, 1, 3, 2).reshape(n_rows, n_cols, N_BINS)


# speedup vs baseline: 20.3790x; 2.4742x over previous
import functools
import math

import jax
import jax.numpy as jnp
import numpy as np
from jax import lax
from jax.experimental import pallas as pl
from jax.experimental.pallas import tpu as pltpu
from jax.experimental.pallas import tpu_sc as plsc

N_BINS = 8
LANES = 16
NC = 2
NS = 16
NW = NC * NS

_TWO_PI = np.float32(2.0 * math.pi)
_INV_BIN = np.float32(N_BINS / (2.0 * math.pi))

_MESH = plsc.VectorSubcoreMesh(core_axis_name="c", subcore_axis_name="s")
_SC_PARAMS = pltpu.CompilerParams(needs_layout_passes=False)


def _worker_id():
    return lax.axis_index("s") * NC + lax.axis_index("c")


def _make_min_kernel(n):
    per_w = n // NW
    chunk = 8192
    nch = per_w // chunk

    @functools.partial(
        pl.kernel,
        out_type=jax.ShapeDtypeStruct((NW, LANES), jnp.float32),
        mesh=_MESH,
        scratch_types=[
            pltpu.VMEM((chunk,), jnp.float32),
            pltpu.VMEM((chunk,), jnp.float32),
            pltpu.VMEM((LANES,), jnp.float32),
            pltpu.SemaphoreType.DMA,
            pltpu.SemaphoreType.DMA,
        ],
        compiler_params=_SC_PARAMS,
    )
    def min_kernel(x_hbm, out_hbm, xb0, xb1, minbuf, s0, s1):
        wid = _worker_id()
        base = wid * per_w
        xbufs = (xb0, xb1)
        sems = (s0, s1)

        def in_slice(c):
            return x_hbm.at[pl.ds(base + c * chunk, chunk)]

        pltpu.async_copy(in_slice(0), xb0, s0)
        pltpu.async_copy(in_slice(1), xb1, s1)

        def gbody(g, acc):
            for b in range(2):
                c = 2 * g + b
                pltpu.make_async_copy(in_slice(c), xbufs[b], sems[b]).wait()

                def vbody(i, a):
                    return jnp.minimum(a, xbufs[b][pl.ds(i * LANES, LANES)])

                acc = lax.fori_loop(0, chunk // LANES, vbody, acc, unroll=8)

                @pl.when(c + 2 < nch)
                def _():
                    pltpu.async_copy(in_slice(c + 2), xbufs[b], sems[b])
            return acc

        acc = lax.fori_loop(
            0, nch // 2, gbody, jnp.full((LANES,), jnp.inf, jnp.float32)
        )
        minbuf[...] = acc
        pltpu.sync_copy(minbuf, out_hbm.at[wid])

    return min_kernel


def _make_encode_kernel(n_rows, n_cols):
    rows_per_w = n_rows // NW
    rch = 2
    cw = rch * n_cols
    ow = cw * N_BINS
    nch = rows_per_w // rch
    n_tiles = n_cols // 128

    @functools.partial(
        pl.kernel,
        out_type=jax.ShapeDtypeStruct((n_rows * n_cols * N_BINS,), jnp.float32),
        mesh=_MESH,
        scratch_types=[
            pltpu.VMEM((cw,), jnp.float32),
            pltpu.VMEM((cw,), jnp.float32),
            pltpu.VMEM((ow,), jnp.float32),
            pltpu.VMEM((ow,), jnp.float32),
            pltpu.VMEM((NW, LANES), jnp.float32),
            pltpu.VMEM((LANES,), jnp.float32),
            pltpu.VMEM((cw,), jnp.int32),
            pltpu.VMEM((cw,), jnp.int32),
            pltpu.SemaphoreType.DMA,
            pltpu.SemaphoreType.DMA,
            pltpu.SemaphoreType.DMA,
            pltpu.SemaphoreType.DMA,
        ],
        compiler_params=_SC_PARAMS,
    )
    def encode_kernel(x_hbm, pmin_hbm, out_hbm,
                      xb0, xb1, ob0, ob1, pbuf, mbuf, ix0, ix1,
                      si0, si1, so0, so1):
        wid = _worker_id()
        in_base = wid * rows_per_w * n_cols
        out_base = in_base * N_BINS
        xbufs = (xb0, xb1)
        obufs = (ob0, ob1)
        ixbufs = (ix0, ix1)
        isems = (si0, si1)
        osems = (so0, so1)

        pltpu.sync_copy(pmin_hbm, pbuf)

        def mb(i, a):
            return jnp.minimum(a, pbuf[i])

        acc = lax.fori_loop(0, NW, mb, jnp.full((LANES,), jnp.inf, jnp.float32))
        lane = lax.iota(jnp.int32, LANES)
        ones = jnp.full((LANES,), 1.0, jnp.float32)
        zeros = jnp.zeros((LANES,), jnp.float32)
        for stride in (8, 4, 2, 1):
            mbuf[...] = acc
            acc = jnp.minimum(acc, plsc.load_gather(mbuf, [lane ^ stride]))
        off = jnp.where(acc < 0.0, ones, zeros)
        eight = jnp.full((LANES,), N_BINS, jnp.int32)

        def in_slice(c):
            return x_hbm.at[pl.ds(in_base + c * cw, cw)]

        def out_slice(c):
            return out_hbm.at[pl.ds(out_base + c * ow, ow)]

        pltpu.async_copy(in_slice(0), xb0, si0)
        pltpu.async_copy(in_slice(1), xb1, si1)

        for ob in obufs:
            @plsc.parallel_loop(0, ow // LANES, unroll=8)
            def _(z):
                ob[pl.ds(z * LANES, LANES)] = zeros
        for ix in ixbufs:
            @plsc.parallel_loop(0, cw // LANES, unroll=8)
            def _(z):
                ix[pl.ds(z * LANES, LANES)] = lane

        def compute(xb, ob, ix):
            @plsc.parallel_loop(0, cw // LANES, unroll=8)
            def _(v):
                old = ix[pl.ds(v * LANES, LANES)]
                plsc.store_scatter(ob, [old], zeros)

            @plsc.parallel_loop(0, cw // LANES, unroll=8)
            def _(v):
                xv = xb[pl.ds(v * LANES, LANES)]
                xn = xv + off
                r0 = lax.rem(xn, _TWO_PI)
                rr = jnp.where(r0 < 0.0, r0 + _TWO_PI, r0)
                bv = jnp.minimum((rr * _INV_BIN).astype(jnp.int32), N_BINS - 1)
                valid = rr < _TWO_PI
                ob_base = ((v >> 3) << 10) + ((v & 7) << 4)
                idx = (bv << 7) + (lane + ob_base)
                plsc.store_scatter(ob, [idx], ones, mask=valid)
                ix[pl.ds(v * LANES, LANES)] = idx

        def gbody(g, _):
            for b in range(2):
                c = 2 * g + b
                pltpu.make_async_copy(in_slice(c), xbufs[b], isems[b]).wait()

                @pl.when(g > 0)
                def _():
                    pltpu.make_async_copy(obufs[b], out_slice(c), osems[b]).wait()

                compute(xbufs[b], obufs[b], ixbufs[b])
                pltpu.async_copy(obufs[b], out_slice(c), osems[b])

                @pl.when(c + 2 < nch)
                def _():
                    pltpu.async_copy(in_slice(c + 2), xbufs[b], isems[b])
            return 0

        lax.fori_loop(0, nch // 2, gbody, 0)
        pltpu.make_async_copy(ob0, out_slice(0), so0).wait()
        pltpu.make_async_copy(ob1, out_slice(1), so1).wait()

    return encode_kernel


def kernel(x, phase_bins):
    del phase_bins
    n_rows, n_cols = x.shape
    xf = x.reshape(-1)
    pmin = _make_min_kernel(x.size)(xf)
    outf = _make_encode_kernel(n_rows, n_cols)(xf, pmin)
    out4 = outf.reshape(n_rows, n_cols // 128, N_BINS, 128)
    return out4.transpose(0, 1, 3, 2).reshape(n_rows, n_cols, N_BINS)


# speedup vs baseline: 27.1563x; 1.3326x over previous
import functools
import math

import jax
import jax.numpy as jnp
import numpy as np
from jax import lax
from jax.experimental import pallas as pl
from jax.experimental.pallas import tpu as pltpu
from jax.experimental.pallas import tpu_sc as plsc

N_BINS = 8
LANES = 16
NC = 2
NS = 16
NW = NC * NS

_TWO_PI = np.float32(2.0 * math.pi)
_INV_BIN = np.float32(N_BINS / (2.0 * math.pi))

_MESH = plsc.VectorSubcoreMesh(core_axis_name="c", subcore_axis_name="s")
_SC_PARAMS = pltpu.CompilerParams(needs_layout_passes=False)


def _worker_id():
    return lax.axis_index("s") * NC + lax.axis_index("c")


def _make_min_kernel(n):
    per_w = n // NW
    chunk = 8192
    nch = per_w // chunk

    @functools.partial(
        pl.kernel,
        out_type=jax.ShapeDtypeStruct((NW, LANES), jnp.float32),
        mesh=_MESH,
        scratch_types=[
            pltpu.VMEM((chunk,), jnp.float32),
            pltpu.VMEM((LANES,), jnp.float32),
        ],
        compiler_params=_SC_PARAMS,
    )
    def min_kernel(x_hbm, out_hbm, xb, minbuf):
        wid = _worker_id()
        base = wid * per_w
        lane = lax.iota(jnp.int32, LANES)
        ones = jnp.full((LANES,), 1.0, jnp.float32)

        def cond(carry):
            c, found = carry
            return jnp.logical_and(c < nch, jnp.logical_not(found))

        def body(carry):
            c, found = carry
            pltpu.sync_copy(x_hbm.at[pl.ds(base + c * chunk, chunk)], xb)

            def vbody(i, a):
                return jnp.minimum(a, xb[pl.ds(i * LANES, LANES)])

            acc = lax.fori_loop(
                0, chunk // LANES, vbody,
                jnp.full((LANES,), jnp.inf, jnp.float32), unroll=8,
            )
            for stride in (8, 4, 2, 1):
                minbuf[...] = acc
                acc = jnp.minimum(acc, plsc.load_gather(minbuf, [lane ^ stride]))
            return c + 1, acc[0] < 0.0

        _, found = lax.while_loop(cond, body, (jnp.int32(0), jnp.bool_(False)))
        minbuf[...] = jnp.where(found, -ones, ones)
        pltpu.sync_copy(minbuf, out_hbm.at[wid])

    return min_kernel


def _make_encode_kernel(n_rows, n_cols):
    rows_per_w = n_rows // NW
    rch = 2
    cw = rch * n_cols
    ow = cw * N_BINS
    nch = rows_per_w // rch
    n_tiles = n_cols // 128

    @functools.partial(
        pl.kernel,
        out_type=jax.ShapeDtypeStruct((n_rows * n_cols * N_BINS,), jnp.float32),
        mesh=_MESH,
        scratch_types=[
            pltpu.VMEM((rch * n_tiles, 128), jnp.float32),
            pltpu.VMEM((rch * n_tiles, 128), jnp.float32),
            pltpu.VMEM((ow,), jnp.float32),
            pltpu.VMEM((ow,), jnp.float32),
            pltpu.VMEM((NW, LANES), jnp.float32),
            pltpu.VMEM((LANES,), jnp.float32),
            pltpu.VMEM((cw,), jnp.int32),
            pltpu.VMEM((cw,), jnp.int32),
            pltpu.SemaphoreType.DMA,
            pltpu.SemaphoreType.DMA,
            pltpu.SemaphoreType.DMA,
            pltpu.SemaphoreType.DMA,
        ],
        compiler_params=_SC_PARAMS,
    )
    def encode_kernel(x_hbm, pmin_hbm, out_hbm,
                      xb0, xb1, ob0, ob1, pbuf, mbuf, ix0, ix1,
                      si0, si1, so0, so1):
        wid = _worker_id()
        row0 = wid * rows_per_w
        out_base = row0 * n_cols * N_BINS
        xbufs = (xb0, xb1)
        obufs = (ob0, ob1)
        ixbufs = (ix0, ix1)
        isems = (si0, si1)
        osems = (so0, so1)

        pltpu.sync_copy(pmin_hbm, pbuf)

        def mb(i, a):
            return jnp.minimum(a, pbuf[i])

        acc = lax.fori_loop(0, NW, mb, jnp.full((LANES,), jnp.inf, jnp.float32))
        lane = lax.iota(jnp.int32, LANES)
        ones = jnp.full((LANES,), 1.0, jnp.float32)
        zeros = jnp.zeros((LANES,), jnp.float32)
        for stride in (8, 4, 2, 1):
            mbuf[...] = acc
            acc = jnp.minimum(acc, plsc.load_gather(mbuf, [lane ^ stride]))
        off = jnp.where(acc < 0.0, ones, zeros)
        eight = jnp.full((LANES,), N_BINS, jnp.int32)

        def start_in(c, xb, sem):
            for ri in range(rch):
                r = row0 + c * rch + ri
                pltpu.async_copy(
                    x_hbm.at[r >> 3, :, r & 7],
                    xb.at[pl.ds(ri * n_tiles, n_tiles)],
                    sem,
                )

        def wait_in(c, xb, sem):
            for ri in range(rch):
                pltpu.make_async_copy(
                    x_hbm.at[0, :, 0],
                    xb.at[pl.ds(ri * n_tiles, n_tiles)],
                    sem,
                ).wait()

        def out_slice(c):
            return out_hbm.at[pl.ds(out_base + c * ow, ow)]

        start_in(0, xb0, si0)
        start_in(1, xb1, si1)

        for ob in obufs:
            @plsc.parallel_loop(0, ow // LANES, unroll=8)
            def _(z):
                ob[pl.ds(z * LANES, LANES)] = zeros
        for ix in ixbufs:
            @plsc.parallel_loop(0, cw // LANES, unroll=8)
            def _(z):
                ix[pl.ds(z * LANES, LANES)] = lane

        def compute(xb, ob, ix):
            @plsc.parallel_loop(0, cw // LANES, unroll=8)
            def _(v):
                old = ix[pl.ds(v * LANES, LANES)]
                plsc.store_scatter(ob, [old], zeros)

            @plsc.parallel_loop(0, cw // LANES, unroll=8)
            def _(v):
                xv = xb[v >> 3, pl.ds((v & 7) << 4, LANES)]
                xn = xv + off
                r0 = lax.rem(xn, _TWO_PI)
                rr = jnp.where(r0 < 0.0, r0 + _TWO_PI, r0)
                bv = jnp.minimum((rr * _INV_BIN).astype(jnp.int32), N_BINS - 1)
                valid = rr < _TWO_PI
                ob_base = ((v >> 3) << 10) + ((v & 7) << 4)
                idx = (bv << 7) + (lane + ob_base)
                plsc.store_scatter(ob, [idx], ones, mask=valid)
                ix[pl.ds(v * LANES, LANES)] = idx

        def gbody(g, _):
            for b in range(2):
                c = 2 * g + b
                wait_in(c, xbufs[b], isems[b])

                @pl.when(g > 0)
                def _():
                    pltpu.make_async_copy(obufs[b], out_slice(c), osems[b]).wait()

                compute(xbufs[b], obufs[b], ixbufs[b])
                pltpu.async_copy(obufs[b], out_slice(c), osems[b])

                @pl.when(c + 2 < nch)
                def _():
                    start_in(c + 2, xbufs[b], isems[b])
            return 0

        lax.fori_loop(0, nch // 2, gbody, 0)
        pltpu.make_async_copy(ob0, out_slice(0), so0).wait()
        pltpu.make_async_copy(ob1, out_slice(1), so1).wait()

    return encode_kernel


def kernel(x, phase_bins):
    del phase_bins
    n_rows, n_cols = x.shape
    xnat = (
        x.reshape(n_rows // 8, 8, n_cols // 128, 128).transpose(0, 2, 1, 3)
    )
    xnat_flat = xnat.reshape(-1)
    pmin = _make_min_kernel(x.size)(xnat_flat)
    outf = _make_encode_kernel(n_rows, n_cols)(xnat, pmin)
    out4 = outf.reshape(n_rows, n_cols // 128, N_BINS, 128)
    return out4.transpose(0, 1, 3, 2).reshape(n_rows, n_cols, N_BINS)


# speedup vs baseline: 27.8637x; 1.0261x over previous
import functools
import math

import jax
import jax.numpy as jnp
import numpy as np
from jax import lax
from jax.experimental import pallas as pl
from jax.experimental.pallas import tpu as pltpu
from jax.experimental.pallas import tpu_sc as plsc

N_BINS = 8
LANES = 16
NC = 2
NS = 16
NW = NC * NS

_TWO_PI = np.float32(2.0 * math.pi)
_INV_BIN = np.float32(N_BINS / (2.0 * math.pi))

_MESH = plsc.VectorSubcoreMesh(core_axis_name="c", subcore_axis_name="s")
_SC_PARAMS = pltpu.CompilerParams(needs_layout_passes=False)


def _worker_id():
    return lax.axis_index("s") * NC + lax.axis_index("c")


def _make_encode_kernel(n_rows, n_cols):
    rows_per_w = n_rows // NW
    rch = 2
    cw = rch * n_cols
    ow = cw * N_BINS
    nch = rows_per_w // rch
    n_tiles = n_cols // 128

    @functools.partial(
        pl.kernel,
        out_type=jax.ShapeDtypeStruct((n_rows * n_cols * N_BINS,), jnp.float32),
        mesh=_MESH,
        scratch_types=[
            pltpu.VMEM((rch * n_tiles, 128), jnp.float32),
            pltpu.VMEM((rch * n_tiles, 128), jnp.float32),
            pltpu.VMEM((ow,), jnp.float32),
            pltpu.VMEM((ow,), jnp.float32),
            pltpu.VMEM((16, 8, 128), jnp.float32),
            pltpu.VMEM((LANES,), jnp.float32),
            pltpu.VMEM((cw,), jnp.int32),
            pltpu.VMEM((cw,), jnp.int32),
            pltpu.SemaphoreType.DMA,
            pltpu.SemaphoreType.DMA,
            pltpu.SemaphoreType.DMA,
            pltpu.SemaphoreType.DMA,
        ],
        compiler_params=_SC_PARAMS,
    )
    def encode_kernel(x_hbm, out_hbm,
                      xb0, xb1, ob0, ob1, sbuf, mbuf, ix0, ix1,
                      si0, si1, so0, so1):
        wid = _worker_id()
        row0 = wid * rows_per_w
        out_base = row0 * n_cols * N_BINS
        xbufs = (xb0, xb1)
        obufs = (ob0, ob1)
        ixbufs = (ix0, ix1)
        isems = (si0, si1)
        osems = (so0, so1)

        lane = lax.iota(jnp.int32, LANES)
        ones = jnp.full((LANES,), 1.0, jnp.float32)
        zeros = jnp.zeros((LANES,), jnp.float32)

        scan_chunk = 16 * 8 * 128
        n_scan = (n_rows * n_cols) // scan_chunk

        def scond(carry):
            c, found = carry
            return jnp.logical_and(c < n_scan, jnp.logical_not(found))

        def sbody(carry):
            c, found = carry
            pltpu.sync_copy(x_hbm.at[c], sbuf)

            def vbody(i, a):
                return jnp.minimum(
                    a, sbuf[i >> 6, (i >> 3) & 7, pl.ds((i & 7) << 4, LANES)]
                )

            acc = lax.fori_loop(
                0, scan_chunk // LANES, vbody,
                jnp.full((LANES,), jnp.inf, jnp.float32), unroll=8,
            )
            for stride in (8, 4, 2, 1):
                mbuf[...] = acc
                acc = jnp.minimum(acc, plsc.load_gather(mbuf, [lane ^ stride]))
            return c + 1, acc[0] < 0.0

        _, neg = lax.while_loop(scond, sbody, (jnp.int32(0), jnp.bool_(False)))
        off = jnp.where(neg, ones, zeros)

        def start_in(c, xb, sem):
            for ri in range(rch):
                r = row0 + c * rch + ri
                pltpu.async_copy(
                    x_hbm.at[r >> 3, :, r & 7],
                    xb.at[pl.ds(ri * n_tiles, n_tiles)],
                    sem,
                )

        def wait_in(c, xb, sem):
            for ri in range(rch):
                pltpu.make_async_copy(
                    x_hbm.at[0, :, 0],
                    xb.at[pl.ds(ri * n_tiles, n_tiles)],
                    sem,
                ).wait()

        def out_slice(c):
            return out_hbm.at[pl.ds(out_base + c * ow, ow)]

        start_in(0, xb0, si0)
        start_in(1, xb1, si1)

        for ob in obufs:
            @plsc.parallel_loop(0, ow // LANES, unroll=8)
            def _(z):
                ob[pl.ds(z * LANES, LANES)] = zeros
        for ix in ixbufs:
            @plsc.parallel_loop(0, cw // LANES, unroll=8)
            def _(z):
                ix[pl.ds(z * LANES, LANES)] = lane

        def compute(xb, ob, ix):
            @plsc.parallel_loop(0, cw // LANES, unroll=8)
            def _(v):
                old = ix[pl.ds(v * LANES, LANES)]
                plsc.store_scatter(ob, [old], zeros)

            @plsc.parallel_loop(0, cw // LANES, unroll=8)
            def _(v):
                xv = xb[v >> 3, pl.ds((v & 7) << 4, LANES)]
                xn = xv + off
                r0 = lax.rem(xn, _TWO_PI)
                rr = jnp.where(r0 < 0.0, r0 + _TWO_PI, r0)
                bv = jnp.minimum((rr * _INV_BIN).astype(jnp.int32), N_BINS - 1)
                valid = rr < _TWO_PI
                ob_base = ((v >> 3) << 10) + ((v & 7) << 4)
                idx = (bv << 7) + (lane + ob_base)
                plsc.store_scatter(ob, [idx], ones, mask=valid)
                ix[pl.ds(v * LANES, LANES)] = idx

        def gbody(g, _):
            for b in range(2):
                c = 2 * g + b
                wait_in(c, xbufs[b], isems[b])

                @pl.when(g > 0)
                def _():
                    pltpu.make_async_copy(obufs[b], out_slice(c), osems[b]).wait()

                compute(xbufs[b], obufs[b], ixbufs[b])
                pltpu.async_copy(obufs[b], out_slice(c), osems[b])

                @pl.when(c + 2 < nch)
                def _():
                    start_in(c + 2, xbufs[b], isems[b])
            return 0

        lax.fori_loop(0, nch // 2, gbody, 0)
        pltpu.make_async_copy(ob0, out_slice(0), so0).wait()
        pltpu.make_async_copy(ob1, out_slice(1), so1).wait()

    return encode_kernel


def kernel(x, phase_bins):
    del phase_bins
    n_rows, n_cols = x.shape
    xnat = (
        x.reshape(n_rows // 8, 8, n_cols // 128, 128).transpose(0, 2, 1, 3)
    )
    outf = _make_encode_kernel(n_rows, n_cols)(xnat)
    out4 = outf.reshape(n_rows, n_cols // 128, N_BINS, 128)
    return out4.transpose(0, 1, 3, 2).reshape(n_rows, n_cols, N_BINS)
